# baseline (speedup 1.0000x reference)
import jax
import jax.numpy as jnp
from jax.experimental import pallas as pl
from jax.experimental.pallas import tpu as pltpu

_N = 4096
_NF = 2048
_TI = 1024
_NT = _N // _TI
_NTF = _NF // _TI
_K = 16
_NC = 1024


def _split3(x):
    b1 = x.astype(jnp.bfloat16).astype(jnp.float32)
    r1 = x - b1
    b2 = r1.astype(jnp.bfloat16).astype(jnp.float32)
    b3 = (r1 - b2).astype(jnp.bfloat16).astype(jnp.float32)
    return b1, b2, b3


def _augment(pts, coord_scale, splits_first):
    f32 = jnp.float32
    bf16 = jnp.bfloat16
    c0 = pts[0:1, :]
    c1 = pts[1:2, :]
    c2 = pts[2:3, :]
    nrm = c0 * c0 + c1 * c1 + c2 * c2
    s1, s2, s3 = _split3(nrm)
    ones_r = jnp.ones((1, _N), f32)
    zeros_r = jnp.zeros((_K - 9, _N), f32)
    mid = [s1, s2, s3, ones_r, ones_r, ones_r]
    if not splits_first:
        mid = [ones_r, ones_r, ones_r, s1, s2, s3]
    return jnp.concatenate(
        [
            coord_scale * c0.astype(bf16).astype(f32),
            coord_scale * c1.astype(bf16).astype(f32),
            coord_scale * c2.astype(bf16).astype(f32),
        ]
        + mid
        + [zeros_r],
        axis=0,
    ).astype(bf16)


def _cham_kernel(pred_ref, gt_ref, out_ref, a_ref):
    f32 = jnp.float32

    predt = jnp.transpose(pred_ref[0])
    gtt = jnp.transpose(gt_ref[0])
    a_ref[...] = _augment(predt * 80.0, jnp.float32(1.0), False)
    g_aug = _augment(gtt * 80.0, jnp.float32(-2.0), True)

    inf = jnp.float32(jnp.inf)
    zero = jnp.float32(0.0)

    cm0 = jnp.full((1, _NF), inf, f32)
    cml_h, cml_l, cmr, srf, srh = cm0, cm0, cm0, zero, zero
    for t in range(_NT):
        i0 = t * _TI
        a_t = a_ref[:, i0:i0 + _TI]
        d = jax.lax.dot_general(
            a_t, g_aug, (((0,), (0,)), ((), ())),
            preferred_element_type=f32,
        )
        dl = d[:, :_NF]
        dr = d[:, _NF:]
        rml = jnp.min(dl, axis=1, keepdims=True)
        rmr = jnp.min(dr, axis=1, keepdims=True)
        srf = srf + jnp.sum(jnp.maximum(jnp.minimum(rml, rmr), zero))
        cl = jnp.min(dl, axis=0, keepdims=True)
        cr = jnp.min(dr, axis=0, keepdims=True)
        if t < _NTF:
            srh = srh + jnp.sum(jnp.maximum(rml, zero))
            cml_h = jnp.minimum(cml_h, cl)
        else:
            cml_l = jnp.minimum(cml_l, cl)
        cmr = jnp.minimum(cmr, cr)
    scf = jnp.sum(jnp.maximum(jnp.minimum(cml_h, cml_l), zero)) + jnp.sum(
        jnp.maximum(cmr, zero)
    )
    sch = jnp.sum(jnp.maximum(cml_h, zero))

    def row(s):
        return jnp.full((1, 128), s, f32)

    out_ref[0] = jnp.concatenate(
        [row(srf), row(scf), row(srh), row(sch)], axis=0
    )


def kernel(image_pred, image_gt):
    b = image_pred.shape[0]
    out = pl.pallas_call(
        _cham_kernel,
        grid=(b,),
        in_specs=[
            pl.BlockSpec((1, _N, 3), lambda i: (i, 0, 0)),
            pl.BlockSpec((1, _N, 3), lambda i: (i, 0, 0)),
        ],
        out_specs=pl.BlockSpec((1, 4, 128), lambda i: (i, 0, 0)),
        out_shape=jax.ShapeDtypeStruct((b, 4, 128), jnp.float32),
        scratch_shapes=[pltpu.VMEM((_K, _N), jnp.bfloat16)],
    )(image_pred, image_gt)
    v = out[:, :, 0]
    cham_full = (v[:, 0] + v[:, 1]) / _N
    cham_half = (v[:, 2] + v[:, 3]) / _NF
    return 0.7 * jnp.mean(cham_half) + 0.3 * jnp.mean(cham_full)


# speedup vs baseline: 1.1286x; 1.1286x over previous
import jax
import jax.numpy as jnp
from jax.experimental import pallas as pl
from jax.experimental.pallas import tpu as pltpu

_B = 4
_N = 4096
_NF = 2048
_TI = 1024
_NT = _N // _TI
_NTF = _NF // _TI
_K = 16


def _split3(x):
    b1 = x.astype(jnp.bfloat16).astype(jnp.float32)
    r1 = x - b1
    b2 = r1.astype(jnp.bfloat16).astype(jnp.float32)
    b3 = (r1 - b2).astype(jnp.bfloat16).astype(jnp.float32)
    return b1, b2, b3


def _augment(pts, coord_scale, splits_first):
    f32 = jnp.float32
    bf16 = jnp.bfloat16
    c0 = pts[0:1, :]
    c1 = pts[1:2, :]
    c2 = pts[2:3, :]
    nrm = c0 * c0 + c1 * c1 + c2 * c2
    s1, s2, s3 = _split3(nrm)
    ones_r = jnp.ones((1, _N), f32)
    zeros_r = jnp.zeros((_K - 9, _N), f32)
    mid = [s1, s2, s3, ones_r, ones_r, ones_r]
    if not splits_first:
        mid = [ones_r, ones_r, ones_r, s1, s2, s3]
    return jnp.concatenate(
        [
            coord_scale * c0.astype(bf16).astype(f32),
            coord_scale * c1.astype(bf16).astype(f32),
            coord_scale * c2.astype(bf16).astype(f32),
        ]
        + mid
        + [zeros_r],
        axis=0,
    ).astype(bf16)


def _cham_kernel(predt_ref, gtt_ref, out_ref, a_ref):
    f32 = jnp.float32

    inf = jnp.float32(jnp.inf)
    zero = jnp.float32(0.0)
    srf_t, scf_t, srh_t, sch_t = zero, zero, zero, zero

    for b in range(_B):
        a_ref[...] = _augment(predt_ref[b] * 80.0, jnp.float32(1.0), False)
        g_aug = _augment(gtt_ref[b] * 80.0, jnp.float32(-2.0), True)

        cm0 = jnp.full((1, _NF), inf, f32)
        cml_h, cml_l, cmr, srf, srh = cm0, cm0, cm0, zero, zero
        for t in range(_NT):
            i0 = t * _TI
            a_t = a_ref[:, i0:i0 + _TI]
            d = jax.lax.dot_general(
                a_t, g_aug, (((0,), (0,)), ((), ())),
                preferred_element_type=f32,
            )
            dl = d[:, :_NF]
            dr = d[:, _NF:]
            rml = jnp.min(dl, axis=1, keepdims=True)
            rmr = jnp.min(dr, axis=1, keepdims=True)
            srf = srf + jnp.sum(jnp.maximum(jnp.minimum(rml, rmr), zero))
            cl = jnp.min(dl, axis=0, keepdims=True)
            cr = jnp.min(dr, axis=0, keepdims=True)
            if t < _NTF:
                srh = srh + jnp.sum(jnp.maximum(rml, zero))
                cml_h = jnp.minimum(cml_h, cl)
            else:
                cml_l = jnp.minimum(cml_l, cl)
            cmr = jnp.minimum(cmr, cr)
        scf = jnp.sum(
            jnp.maximum(jnp.minimum(cml_h, cml_l), zero)
        ) + jnp.sum(jnp.maximum(cmr, zero))
        sch = jnp.sum(jnp.maximum(cml_h, zero))
        srf_t = srf_t + srf
        scf_t = scf_t + scf
        srh_t = srh_t + srh
        sch_t = sch_t + sch

    def row(s):
        return jnp.full((1, 128), s, f32)

    out_ref[...] = jnp.concatenate(
        [row(srf_t), row(scf_t), row(srh_t), row(sch_t)], axis=0
    )


def kernel(image_pred, image_gt):
    predt = jnp.transpose(image_pred, (0, 2, 1))
    gtt = jnp.transpose(image_gt, (0, 2, 1))
    out = pl.pallas_call(
        _cham_kernel,
        out_shape=jax.ShapeDtypeStruct((4, 128), jnp.float32),
        scratch_shapes=[pltpu.VMEM((_K, _N), jnp.bfloat16)],
    )(predt, gtt)
    v = out[:, 0]
    cham_full = (v[0] + v[1]) / _N
    cham_half = (v[2] + v[3]) / _NF
    return 0.7 * cham_half / _B + 0.3 * cham_full / _B


# speedup vs baseline: 1.2854x; 1.1390x over previous
import jax
import jax.numpy as jnp
from jax.experimental import pallas as pl
from jax.experimental.pallas import tpu as pltpu

_N = 4096
_NF = 2048
_TI = 1024
_NT = _N // _TI
_NTF = _NF // _TI
_K = 16


def _split3(x):
    b1 = x.astype(jnp.bfloat16).astype(jnp.float32)
    r1 = x - b1
    b2 = r1.astype(jnp.bfloat16).astype(jnp.float32)
    b3 = (r1 - b2).astype(jnp.bfloat16).astype(jnp.float32)
    return b1, b2, b3


def _augment(pts, coord_scale, splits_first):
    f32 = jnp.float32
    bf16 = jnp.bfloat16
    c0 = pts[0:1, :]
    c1 = pts[1:2, :]
    c2 = pts[2:3, :]
    nrm = c0 * c0 + c1 * c1 + c2 * c2
    s1, s2, s3 = _split3(nrm)
    ones_r = jnp.ones((1, _N), f32)
    zeros_r = jnp.zeros((_K - 9, _N), f32)
    mid = [s1, s2, s3, ones_r, ones_r, ones_r]
    if not splits_first:
        mid = [ones_r, ones_r, ones_r, s1, s2, s3]
    return jnp.concatenate(
        [
            coord_scale * c0.astype(bf16).astype(f32),
            coord_scale * c1.astype(bf16).astype(f32),
            coord_scale * c2.astype(bf16).astype(f32),
        ]
        + mid
        + [zeros_r],
        axis=0,
    ).astype(bf16)


def _cham_kernel(predt_ref, gtt_ref, out_ref, a_ref):
    f32 = jnp.float32

    a_ref[...] = _augment(predt_ref[0] * 80.0, jnp.float32(1.0), False)
    g_aug = _augment(gtt_ref[0] * 80.0, jnp.float32(-2.0), True)

    inf = jnp.float32(jnp.inf)
    zero = jnp.float32(0.0)

    cm0 = jnp.full((1, _NF), inf, f32)
    cml_h, cml_l, cmr, srf, srh = cm0, cm0, cm0, zero, zero
    for t in range(_NT):
        i0 = t * _TI
        a_t = a_ref[:, i0:i0 + _TI]
        d = jax.lax.dot_general(
            a_t, g_aug, (((0,), (0,)), ((), ())),
            preferred_element_type=f32,
        )
        dl = d[:, :_NF]
        dr = d[:, _NF:]
        rml = jnp.min(dl, axis=1, keepdims=True)
        rmr = jnp.min(dr, axis=1, keepdims=True)
        srf = srf + jnp.sum(jnp.maximum(jnp.minimum(rml, rmr), zero))
        cl = jnp.min(dl, axis=0, keepdims=True)
        cr = jnp.min(dr, axis=0, keepdims=True)
        if t < _NTF:
            srh = srh + jnp.sum(jnp.maximum(rml, zero))
            cml_h = jnp.minimum(cml_h, cl)
        else:
            cml_l = jnp.minimum(cml_l, cl)
        cmr = jnp.minimum(cmr, cr)
    scf = jnp.sum(jnp.maximum(jnp.minimum(cml_h, cml_l), zero)) + jnp.sum(
        jnp.maximum(cmr, zero)
    )
    sch = jnp.sum(jnp.maximum(cml_h, zero))

    def row(s):
        return jnp.full((1, 128), s, f32)

    out_ref[0] = jnp.concatenate(
        [row(srf), row(scf), row(srh), row(sch)], axis=0
    )


def kernel(image_pred, image_gt):
    b = image_pred.shape[0]
    predt = jnp.transpose(image_pred, (0, 2, 1))
    gtt = jnp.transpose(image_gt, (0, 2, 1))
    out = pl.pallas_call(
        _cham_kernel,
        grid=(b,),
        in_specs=[
            pl.BlockSpec((1, 3, _N), lambda i: (i, 0, 0)),
            pl.BlockSpec((1, 3, _N), lambda i: (i, 0, 0)),
        ],
        out_specs=pl.BlockSpec((1, 4, 128), lambda i: (i, 0, 0)),
        out_shape=jax.ShapeDtypeStruct((b, 4, 128), jnp.float32),
        scratch_shapes=[pltpu.VMEM((_K, _N), jnp.bfloat16)],
    )(predt, gtt)
    v = out[:, :, 0]
    cham_full = (v[:, 0] + v[:, 1]) / _N
    cham_half = (v[:, 2] + v[:, 3]) / _NF
    return 0.7 * jnp.mean(cham_half) + 0.3 * jnp.mean(cham_full)


# speedup vs baseline: 1.3932x; 1.0839x over previous
import jax
import jax.numpy as jnp
from jax.experimental import pallas as pl
from jax.experimental.pallas import tpu as pltpu

_N = 4096
_NF = 2048
_TI = 1024
_NT = _N // _TI
_NTF = _NF // _TI
_K = 16


def _split3(x):
    b1 = x.astype(jnp.bfloat16).astype(jnp.float32)
    r1 = x - b1
    b2 = r1.astype(jnp.bfloat16).astype(jnp.float32)
    b3 = (r1 - b2).astype(jnp.bfloat16).astype(jnp.float32)
    return b1, b2, b3


def _augment(pts, coord_scale, splits_first):
    f32 = jnp.float32
    bf16 = jnp.bfloat16
    c0 = pts[0:1, :]
    c1 = pts[1:2, :]
    c2 = pts[2:3, :]
    nrm = c0 * c0 + c1 * c1 + c2 * c2
    s1, s2, s3 = _split3(nrm)
    ones_r = jnp.ones((1, _N), f32)
    zeros_r = jnp.zeros((_K - 9, _N), f32)
    mid = [s1, s2, s3, ones_r, ones_r, ones_r]
    if not splits_first:
        mid = [ones_r, ones_r, ones_r, s1, s2, s3]
    return jnp.concatenate(
        [
            coord_scale * c0.astype(bf16).astype(f32),
            coord_scale * c1.astype(bf16).astype(f32),
            coord_scale * c2.astype(bf16).astype(f32),
        ]
        + mid
        + [zeros_r],
        axis=0,
    ).astype(bf16)


def _cham_kernel(predt_ref, gtt_ref, out_ref, a_ref, acc_ref):
    f32 = jnp.float32
    nb = pl.num_programs(0)
    bi = pl.program_id(0)

    a_ref[...] = _augment(predt_ref[0] * 80.0, jnp.float32(1.0), False)
    g_aug = _augment(gtt_ref[0] * 80.0, jnp.float32(-2.0), True)

    inf = jnp.float32(jnp.inf)
    zero = jnp.float32(0.0)

    cm0 = jnp.full((1, _NF), inf, f32)
    cml_h, cml_l, cmr, srf, srh = cm0, cm0, cm0, zero, zero
    for t in range(_NT):
        i0 = t * _TI
        a_t = a_ref[:, i0:i0 + _TI]
        d = jax.lax.dot_general(
            a_t, g_aug, (((0,), (0,)), ((), ())),
            preferred_element_type=f32,
        )
        dl = d[:, :_NF]
        dr = d[:, _NF:]
        rml = jnp.min(dl, axis=1, keepdims=True)
        rmr = jnp.min(dr, axis=1, keepdims=True)
        srf = srf + jnp.sum(jnp.maximum(jnp.minimum(rml, rmr), zero))
        cl = jnp.min(dl, axis=0, keepdims=True)
        cr = jnp.min(dr, axis=0, keepdims=True)
        if t < _NTF:
            srh = srh + jnp.sum(jnp.maximum(rml, zero))
            cml_h = jnp.minimum(cml_h, cl)
        else:
            cml_l = jnp.minimum(cml_l, cl)
        cmr = jnp.minimum(cmr, cr)
    scf = jnp.sum(jnp.maximum(jnp.minimum(cml_h, cml_l), zero)) + jnp.sum(
        jnp.maximum(cmr, zero)
    )
    sch = jnp.sum(jnp.maximum(cml_h, zero))

    def row(s):
        return jnp.full((1, 128), s, f32)

    vals = jnp.concatenate(
        [row(srf), row(scf), row(srh), row(sch)], axis=0
    )

    @pl.when(bi == 0)
    def _():
        acc_ref[...] = vals

    @pl.when(bi != 0)
    def _():
        acc_ref[...] = acc_ref[...] + vals

    @pl.when(bi == nb - 1)
    def _():
        tsrf = acc_ref[0:1, 0:1]
        tscf = acc_ref[1:2, 0:1]
        tsrh = acc_ref[2:3, 0:1]
        tsch = acc_ref[3:4, 0:1]
        fnb = jnp.float32(nb)
        out_ref[...] = ((0.3 / _N) * (tsrf + tscf)
                        + (0.7 / _NF) * (tsrh + tsch)) / fnb


def kernel(image_pred, image_gt):
    b = image_pred.shape[0]
    predt = jnp.transpose(image_pred, (0, 2, 1))
    gtt = jnp.transpose(image_gt, (0, 2, 1))
    out = pl.pallas_call(
        _cham_kernel,
        grid=(b,),
        in_specs=[
            pl.BlockSpec((1, 3, _N), lambda i: (i, 0, 0)),
            pl.BlockSpec((1, 3, _N), lambda i: (i, 0, 0)),
        ],
        out_specs=pl.BlockSpec((1, 1), lambda i: (0, 0)),
        out_shape=jax.ShapeDtypeStruct((1, 1), jnp.float32),
        scratch_shapes=[
            pltpu.VMEM((_K, _N), jnp.bfloat16),
            pltpu.VMEM((4, 128), jnp.float32),
        ],
    )(predt, gtt)
    return out.reshape(())
